# baseline (speedup 1.0000x reference)
import dataclasses
import functools

import jax
import jax.numpy as jnp
from jax import lax
from jax.experimental import pallas as pl
from jax.experimental.pallas import tpu as pltpu
from jax.experimental.pallas import tpu_sc as plsc

N = 10000
E = 320000
D_IN = 128
D_HID = 200
D_OUT = 8

NC = 2
NS = 16
E_PER_W = E // (NC * NS)
CHUNK = 80
NCH = E_PER_W // CHUNK
NBUF = 4
WB_ROWS = 1000
WB_SUBS = N // WB_ROWS

_mesh = plsc.VectorSubcoreMesh(core_axis_name="c", subcore_axis_name="s")

_sc_cp = pltpu.CompilerParams()
if "needs_layout_passes" in pltpu.CompilerParams.__dataclass_fields__:
  _sc_cp = dataclasses.replace(_sc_cp, needs_layout_passes=False)


def _make_sc_agg(nblk):
  NB = 3

  @functools.partial(
      pl.kernel,
      out_type=jax.ShapeDtypeStruct((NC, N, D_IN), jnp.float32),
      mesh=_mesh,
      compiler_params=_sc_cp,
      scratch_types=[
          pltpu.VMEM((NB * CHUNK,), jnp.int32),
          pltpu.VMEM((NB, CHUNK), jnp.int32),
          pltpu.VMEM((NB * CHUNK,), jnp.float32),
          pltpu.VMEM((NB, CHUNK, D_IN), jnp.float32),
          pltpu.VMEM_SHARED((N, D_IN), jnp.float32),
          pltpu.SemaphoreType.DMA((NB,)),
          pltpu.SemaphoreType.DMA((NB,)),
          pltpu.SemaphoreType.DMA((NB,)),
      ],
  )
  def agg(vals_hbm, row_hbm, col_hbm, ew_hbm, zeros_hbm, out_hbm,
          rowbuf, colbuf, ewbuf, gbuf, acc, semi, semg, sems):
    c = lax.axis_index("c")
    s = lax.axis_index("s")
    ebase = (c * NS + s) * E_PER_W

    def idx_issue(j, q):
      off = ebase + j * CHUNK
      pltpu.async_copy(row_hbm.at[pl.ds(off, CHUNK)],
                       rowbuf.at[pl.ds(q * CHUNK, CHUNK)], semi.at[q])
      pltpu.async_copy(col_hbm.at[pl.ds(off, CHUNK)], colbuf.at[q],
                       semi.at[q])
      pltpu.async_copy(ew_hbm.at[pl.ds(off, CHUNK)],
                       ewbuf.at[pl.ds(q * CHUNK, CHUNK)], semi.at[q])

    def idx_wait(q):
      for _ in range(3):
        pltpu.make_async_copy(col_hbm.at[pl.ds(ebase, CHUNK)], colbuf.at[q],
                              semi.at[q]).wait()

    def g_issue(j, q):
      pltpu.async_copy(
          vals_hbm.at[rowbuf.at[pl.ds(q * CHUNK, CHUNK)]], gbuf.at[q],
          semg.at[q])

    def g_wait(q):
      pltpu.make_async_copy(vals_hbm.at[rowbuf.at[pl.ds(q * CHUNK, CHUNK)]],
                            gbuf.at[q], semg.at[q]).wait()

    def s_issue(q):
      pltpu.async_copy(gbuf.at[q], acc.at[colbuf.at[q]], sems.at[q], add=True)

    def s_wait(q):
      pltpu.make_async_copy(gbuf.at[q], acc.at[colbuf.at[q]], sems.at[q]).wait()

    def scale(b):
      gb = gbuf.at[b]

      @pl.loop(0, CHUNK, step=16)
      def _(g):
        for t in range(16):
          e = g + t
          sv = plsc.load_gather(ewbuf, [lax.broadcast(b * CHUNK + e, (16,))])
          for j in range(nblk):
            gb[e, pl.ds(j * 16, 16)] = gb[e, pl.ds(j * 16, 16)] * sv

    @pl.when(s == 0)
    def _():
      pltpu.sync_copy(zeros_hbm, acc)

    plsc.subcore_barrier()

    idx_issue(0, 0)
    idx_issue(1, 1)
    idx_wait(0)
    g_issue(0, 0)

    @pl.loop(0, NCH - 2, step=NB)
    def _(i0):
      for t in range(NB):
        i = i0 + t
        b = t
        bn = (t + 1) % NB
        bp = (t + 2) % NB
        g_wait(b)
        if t == 0:
          @pl.when(i0 >= 1)
          def _():
            s_wait(bp)
        else:
          s_wait(bp)
        idx_wait(bn)
        g_issue(i + 1, bn)
        idx_issue(i + 2, bp)
        scale(b)
        s_issue(b)

    g_wait(0)
    s_wait(2)
    idx_wait(1)
    g_issue(NCH - 1, 1)
    scale(0)
    s_issue(0)

    g_wait(1)
    s_wait(0)
    scale(1)
    s_issue(1)
    s_wait(1)

    plsc.subcore_barrier()

    @pl.when(s < WB_SUBS)
    def _():
      pltpu.sync_copy(acc.at[pl.ds(s * WB_ROWS, WB_ROWS)],
                      out_hbm.at[c].at[pl.ds(s * WB_ROWS, WB_ROWS)])

  return agg


_sc_agg128 = _make_sc_agg(D_IN // 16)

NP = 1250
NPAD = 1280


@functools.partial(
    pl.kernel,
    out_type=jax.ShapeDtypeStruct((NC, NPAD, D_IN), jnp.float32),
    mesh=_mesh,
    compiler_params=_sc_cp,
    scratch_types=[
        pltpu.VMEM((E_PER_W,), jnp.int32),
        pltpu.VMEM((E_PER_W,), jnp.int32),
        pltpu.VMEM((E_PER_W,), jnp.float32),
        pltpu.VMEM((3 * CHUNK,), jnp.int32),
        pltpu.VMEM((3, CHUNK), jnp.int32),
        pltpu.VMEM((3 * CHUNK,), jnp.int32),
        pltpu.VMEM((3 * CHUNK,), jnp.int32),
        pltpu.VMEM((3, CHUNK, D_IN), jnp.float32),
        pltpu.VMEM((3, CHUNK, D_IN), jnp.float32),
        pltpu.VMEM_SHARED((NPAD, D_IN), jnp.float32),
        pltpu.SemaphoreType.DMA((3,)),
        pltpu.SemaphoreType.DMA((3,)),
    ],
)
def _sc_agg16p(vals_hbm, row_hbm, col_hbm, ew_hbm, zeros_hbm, out_hbm,
               rowbuf, colbuf, ewbuf, qgring, qsring, goring, soring,
               gbuf, sbuf, acc, semg, sems):
  c = lax.axis_index("c")
  s = lax.axis_index("s")
  ebase = (c * NS + s) * E_PER_W
  io = lax.iota(jnp.int32, 16)
  zv = jnp.zeros((16,), jnp.float32)

  pltpu.async_copy(row_hbm.at[pl.ds(ebase, E_PER_W)], rowbuf, semg.at[0])
  pltpu.async_copy(col_hbm.at[pl.ds(ebase, E_PER_W)], colbuf, semg.at[0])
  pltpu.async_copy(ew_hbm.at[pl.ds(ebase, E_PER_W)], ewbuf, semg.at[0])

  for b in range(3):
    sb = sbuf.at[b]

    @pl.loop(0, CHUNK)
    def _(e):
      for j in range(D_IN // 16):
        sb[e, pl.ds(j * 16, 16)] = zv

  @pl.when(s == 0)
  def _():
    pltpu.sync_copy(zeros_hbm, acc)

  for _i in range(3):
    pltpu.make_async_copy(row_hbm.at[pl.ds(ebase, E_PER_W)], rowbuf,
                          semg.at[0]).wait()

  plsc.subcore_barrier()

  def derive(j, q):
    qs2 = qsring.at[q]

    @pl.loop(0, CHUNK, step=16)
    def _(g):
      rv = rowbuf[pl.ds(j * CHUNK + g, 16)]
      qgring[pl.ds(q * CHUNK + g, 16)] = lax.shift_right_logical(rv, 3)
      goring[pl.ds(q * CHUNK + g, 16)] = (rv & 7) * 16
      cv = colbuf[pl.ds(j * CHUNK + g, 16)]
      qs2[pl.ds(g, 16)] = lax.shift_right_logical(cv, 3)
      soring[pl.ds(q * CHUNK + g, 16)] = (cv & 7) * 16

  def g_issue(q):
    pltpu.async_copy(vals_hbm.at[qgring.at[pl.ds(q * CHUNK, CHUNK)]],
                     gbuf.at[q], semg.at[q])

  def g_wait(q):
    pltpu.make_async_copy(vals_hbm.at[qgring.at[pl.ds(q * CHUNK, CHUNK)]],
                          gbuf.at[q], semg.at[q]).wait()

  def s_issue(q):
    pltpu.async_copy(sbuf.at[q], acc.at[qsring.at[q]], sems.at[q], add=True)

  def s_wait_restore(q, restore=True):
    pltpu.make_async_copy(sbuf.at[q], acc.at[qsring.at[q]], sems.at[q]).wait()
    if restore:
      sb = sbuf.at[q]

      @pl.loop(0, CHUNK)
      def _(e):
        ev = lax.broadcast(e, (16,))
        sof = plsc.load_gather(soring, [lax.broadcast(q * CHUNK + e, (16,))])
        plsc.store_scatter(sb, [ev, sof + io], zv)

  def scale(i, b):
    gb = gbuf.at[b]
    sb = sbuf.at[b]

    @pl.loop(0, CHUNK)
    def _(e):
      ev = lax.broadcast(e, (16,))
      sv = plsc.load_gather(ewbuf, [lax.broadcast(i * CHUNK + e, (16,))])
      gof = plsc.load_gather(goring, [lax.broadcast(b * CHUNK + e, (16,))])
      v = plsc.load_gather(gb, [ev, gof + io]) * sv
      sof = plsc.load_gather(soring, [lax.broadcast(b * CHUNK + e, (16,))])
      plsc.store_scatter(sb, [ev, sof + io], v)

  derive(0, 0)
  g_issue(0)
  derive(1, 1)

  @pl.loop(0, NCH - 2, step=3)
  def _(i0):
    for t in range(3):
      i = i0 + t
      b = t
      bn = (t + 1) % 3
      bp = (t + 2) % 3
      g_wait(b)
      if t == 0:
        @pl.when(i0 >= 1)
        def _():
          s_wait_restore(bp)
      else:
        s_wait_restore(bp)
      derive(i + 2, bp)
      g_issue(bn)
      scale(i, b)
      s_issue(b)

  g_wait(0)
  s_wait_restore(2)
  g_issue(1)
  scale(NCH - 2, 0)
  s_issue(0)

  g_wait(1)
  s_wait_restore(0, restore=False)
  scale(NCH - 1, 1)
  s_issue(1)
  s_wait_restore(1, restore=False)

  plsc.subcore_barrier()

  pltpu.sync_copy(acc.at[pl.ds(s * (NPAD // NS), NPAD // NS)],
                  out_hbm.at[c].at[pl.ds(s * (NPAD // NS), NPAD // NS)])


DEG_ROWS = 80


@functools.partial(
    pl.kernel,
    out_type=jax.ShapeDtypeStruct((NC, DEG_ROWS, D_IN), jnp.float32),
    mesh=_mesh,
    compiler_params=_sc_cp,
    scratch_types=[
        pltpu.VMEM((E_PER_W,), jnp.int32),
        pltpu.VMEM((E_PER_W,), jnp.float32),
        pltpu.VMEM((DEG_ROWS, D_IN), jnp.float32),
        pltpu.VMEM((DEG_ROWS,), jnp.int32),
        pltpu.VMEM_SHARED((DEG_ROWS, D_IN), jnp.float32),
        pltpu.SemaphoreType.DMA,
    ],
)
def _sc_deg(col_hbm, ew_hbm, zeros_hbm, out_hbm, colbuf, ewbuf, degbuf,
            idxbuf, acc, sem0):
  c = lax.axis_index("c")
  s = lax.axis_index("s")
  ebase = (c * NS + s) * E_PER_W
  io = lax.iota(jnp.int32, 16)
  zv = jnp.zeros((16,), jnp.float32)

  pltpu.async_copy(col_hbm.at[pl.ds(ebase, E_PER_W)], colbuf, sem0)
  pltpu.async_copy(ew_hbm.at[pl.ds(ebase, E_PER_W)], ewbuf, sem0)

  @pl.loop(0, DEG_ROWS)
  def _(e):
    for j in range(D_IN // 16):
      degbuf[e, pl.ds(j * 16, 16)] = zv

  @pl.loop(0, DEG_ROWS, step=16)
  def _(g):
    idxbuf[pl.ds(g, 16)] = io + g

  @pl.when(s == 0)
  def _():
    pltpu.sync_copy(zeros_hbm, acc)

  for _i in range(2):
    pltpu.make_async_copy(col_hbm.at[pl.ds(ebase, E_PER_W)], colbuf,
                          sem0).wait()

  plsc.subcore_barrier()

  @pl.loop(0, E_PER_W, step=16)
  def _(g):
    cv = colbuf[pl.ds(g, 16)]
    wv = ewbuf[pl.ds(g, 16)]
    plsc.addupdate_scatter(
        degbuf, [lax.shift_right_logical(cv, 7), cv & 127], wv)

  pltpu.async_copy(degbuf, acc.at[idxbuf], sem0, add=True)
  pltpu.make_async_copy(degbuf, acc.at[idxbuf], sem0).wait()

  plsc.subcore_barrier()

  @pl.when(s == 0)
  def _():
    pltpu.sync_copy(acc, out_hbm.at[c])


_BLK = 1000


def _dis_from(deg0, deg1):
  deg = deg0 + deg1 + 1.0
  return jnp.where(deg > 0, lax.rsqrt(deg), 0.0)


def _tc_prescale_body(deg0_ref, deg1_ref, x_ref, xt_ref):
  dis = _dis_from(deg0_ref[...], deg1_ref[...])
  xt_ref[...] = x_ref[...] * dis


def _tc_mid_body(deg0_ref, deg1_ref, a0_ref, a1_ref, xt_ref, w1_ref, b1_ref,
                 w2_ref, pt_ref):
  dis = _dis_from(deg0_ref[...], deg1_ref[...])
  a = (a0_ref[...] + a1_ref[...] + xt_ref[...]) * dis
  h = jnp.dot(a, w1_ref[...], preferred_element_type=jnp.float32) + b1_ref[...]
  h = jnp.maximum(h, 0.0)
  p = jnp.dot(h, w2_ref[...], preferred_element_type=jnp.float32)
  pt_ref[...] = p * dis


def _tc_final_body(deg0_ref, deg1_ref, a0_ref, a1_ref, pt_ref, b2_ref, out_ref):
  dis = _dis_from(deg0_ref[...], deg1_ref[...])
  t = (a0_ref[...] + a1_ref[...] + pt_ref[...]) * dis
  out_ref[...] = t[:, :D_OUT] + b2_ref[...]


def _nblock(width):
  return pl.BlockSpec((_BLK, width), lambda i: (i, 0))


def _full(shape):
  return pl.BlockSpec(shape, lambda i: tuple(0 for _ in shape))


def kernel(x, edge_index, edge_attr, W1, b1, W2, b2):
  row = edge_index[0]
  col = edge_index[1]
  ew = edge_attr
  z128 = jnp.zeros((N, D_IN), jnp.float32)
  z80 = jnp.zeros((DEG_ROWS, D_IN), jnp.float32)
  z1280 = jnp.zeros((NPAD, D_IN), jnp.float32)
  W2p = jnp.pad(W2, ((0, 0), (0, 16 - D_OUT)))
  b1r = b1.reshape(1, D_HID)
  b2r = b2.reshape(1, D_OUT)

  degP = _sc_deg(col, ew, z80)
  degflat = degP.reshape(NC, DEG_ROWS * D_IN)[:, :N]
  deg0 = degflat[0].reshape(N, 1)
  deg1 = degflat[1].reshape(N, 1)

  xt = pl.pallas_call(
      _tc_prescale_body,
      grid=(N // _BLK,),
      in_specs=[_nblock(1), _nblock(1), _nblock(D_IN)],
      out_specs=_nblock(D_IN),
      out_shape=jax.ShapeDtypeStruct((N, D_IN), jnp.float32),
  )(deg0, deg1, x)

  acc1 = _sc_agg128(xt, row, col, ew, z128)

  pt16 = pl.pallas_call(
      _tc_mid_body,
      grid=(N // _BLK,),
      in_specs=[_nblock(1), _nblock(1), _nblock(D_IN), _nblock(D_IN),
                _nblock(D_IN), _full((D_IN, D_HID)), _full((1, D_HID)),
                _full((D_HID, 16))],
      out_specs=_nblock(16),
      out_shape=jax.ShapeDtypeStruct((N, 16), jnp.float32),
  )(deg0, deg1, acc1[0], acc1[1], xt, W1, b1r, W2p)

  ptPk = pt16.reshape(NP, D_IN)

  acc2P = _sc_agg16p(ptPk, row, col, ew, z1280)
  acc2 = acc2P.reshape(NC, NPAD * 8, 16)[:, :N]

  out = pl.pallas_call(
      _tc_final_body,
      grid=(N // _BLK,),
      in_specs=[_nblock(1), _nblock(1), _nblock(16), _nblock(16),
                _nblock(16), _full((1, D_OUT))],
      out_specs=_nblock(D_OUT),
      out_shape=jax.ShapeDtypeStruct((N, D_OUT), jnp.float32),
  )(deg0, deg1, acc2[0], acc2[1], pt16, b2r)

  return out


# speedup vs baseline: 1.0061x; 1.0061x over previous
import dataclasses
import functools

import jax
import jax.numpy as jnp
from jax import lax
from jax.experimental import pallas as pl
from jax.experimental.pallas import tpu as pltpu
from jax.experimental.pallas import tpu_sc as plsc

N = 10000
E = 320000
D_IN = 128
D_HID = 200
D_OUT = 8

NC = 2
NS = 16
E_PER_W = E // (NC * NS)
CHUNK = 80
NCH = E_PER_W // CHUNK
NBUF = 4
WB_ROWS = 1000
WB_SUBS = N // WB_ROWS

_mesh = plsc.VectorSubcoreMesh(core_axis_name="c", subcore_axis_name="s")

_sc_cp = pltpu.CompilerParams()
if "needs_layout_passes" in pltpu.CompilerParams.__dataclass_fields__:
  _sc_cp = dataclasses.replace(_sc_cp, needs_layout_passes=False)


def _make_sc_agg(nblk):
  NB = 3

  @functools.partial(
      pl.kernel,
      out_type=jax.ShapeDtypeStruct((NC, N, D_IN), jnp.float32),
      mesh=_mesh,
      compiler_params=_sc_cp,
      scratch_types=[
          pltpu.VMEM((NB * CHUNK,), jnp.int32),
          pltpu.VMEM((NB, CHUNK), jnp.int32),
          pltpu.VMEM((NB * CHUNK,), jnp.float32),
          pltpu.VMEM((NB, CHUNK, D_IN), jnp.float32),
          pltpu.VMEM_SHARED((N, D_IN), jnp.float32),
          pltpu.SemaphoreType.DMA((NB,)),
          pltpu.SemaphoreType.DMA((NB,)),
          pltpu.SemaphoreType.DMA((NB,)),
      ],
  )
  def agg(vals_hbm, row_hbm, col_hbm, ew_hbm, zeros_hbm, out_hbm,
          rowbuf, colbuf, ewbuf, gbuf, acc, semi, semg, sems):
    c = lax.axis_index("c")
    s = lax.axis_index("s")
    ebase = (c * NS + s) * E_PER_W

    def idx_issue(j, q):
      off = ebase + j * CHUNK
      pltpu.async_copy(row_hbm.at[pl.ds(off, CHUNK)],
                       rowbuf.at[pl.ds(q * CHUNK, CHUNK)], semi.at[q])
      pltpu.async_copy(col_hbm.at[pl.ds(off, CHUNK)], colbuf.at[q],
                       semi.at[q])
      pltpu.async_copy(ew_hbm.at[pl.ds(off, CHUNK)],
                       ewbuf.at[pl.ds(q * CHUNK, CHUNK)], semi.at[q])

    def idx_wait(q):
      for _ in range(3):
        pltpu.make_async_copy(col_hbm.at[pl.ds(ebase, CHUNK)], colbuf.at[q],
                              semi.at[q]).wait()

    def g_issue(j, q):
      pltpu.async_copy(
          vals_hbm.at[rowbuf.at[pl.ds(q * CHUNK, CHUNK)]], gbuf.at[q],
          semg.at[q])

    def g_wait(q):
      pltpu.make_async_copy(vals_hbm.at[rowbuf.at[pl.ds(q * CHUNK, CHUNK)]],
                            gbuf.at[q], semg.at[q]).wait()

    def s_issue(q):
      pltpu.async_copy(gbuf.at[q], acc.at[colbuf.at[q]], sems.at[q], add=True)

    def s_wait(q):
      pltpu.make_async_copy(gbuf.at[q], acc.at[colbuf.at[q]], sems.at[q]).wait()

    def scale(b):
      gb = gbuf.at[b]

      @pl.loop(0, CHUNK, step=16)
      def _(g):
        for t in range(16):
          e = g + t
          sv = plsc.load_gather(ewbuf, [lax.broadcast(b * CHUNK + e, (16,))])
          for j in range(nblk):
            gb[e, pl.ds(j * 16, 16)] = gb[e, pl.ds(j * 16, 16)] * sv

    @pl.when(s == 0)
    def _():
      pltpu.sync_copy(zeros_hbm, acc)

    plsc.subcore_barrier()

    idx_issue(0, 0)
    idx_issue(1, 1)
    idx_wait(0)
    g_issue(0, 0)

    @pl.loop(0, NCH - 2, step=NB)
    def _(i0):
      for t in range(NB):
        i = i0 + t
        b = t
        bn = (t + 1) % NB
        bp = (t + 2) % NB
        g_wait(b)
        if t == 0:
          @pl.when(i0 >= 1)
          def _():
            s_wait(bp)
        else:
          s_wait(bp)
        idx_wait(bn)
        g_issue(i + 1, bn)
        idx_issue(i + 2, bp)
        scale(b)
        s_issue(b)

    g_wait(0)
    s_wait(2)
    idx_wait(1)
    g_issue(NCH - 1, 1)
    scale(0)
    s_issue(0)

    g_wait(1)
    s_wait(0)
    scale(1)
    s_issue(1)
    s_wait(1)

    plsc.subcore_barrier()

    @pl.when(s < WB_SUBS)
    def _():
      pltpu.sync_copy(acc.at[pl.ds(s * WB_ROWS, WB_ROWS)],
                      out_hbm.at[c].at[pl.ds(s * WB_ROWS, WB_ROWS)])

  return agg


_sc_agg128 = _make_sc_agg(D_IN // 16)

NP = 1250
NPAD = 1280


@functools.partial(
    pl.kernel,
    out_type=jax.ShapeDtypeStruct((NC, NPAD, D_IN), jnp.float32),
    mesh=_mesh,
    compiler_params=_sc_cp,
    scratch_types=[
        pltpu.VMEM((E_PER_W,), jnp.int32),
        pltpu.VMEM((E_PER_W,), jnp.int32),
        pltpu.VMEM((E_PER_W,), jnp.float32),
        pltpu.VMEM((3 * CHUNK,), jnp.int32),
        pltpu.VMEM((3, CHUNK), jnp.int32),
        pltpu.VMEM((3 * CHUNK,), jnp.int32),
        pltpu.VMEM((3 * CHUNK,), jnp.int32),
        pltpu.VMEM((3, CHUNK, D_IN), jnp.float32),
        pltpu.VMEM((3, CHUNK, D_IN), jnp.float32),
        pltpu.VMEM_SHARED((NPAD, D_IN), jnp.float32),
        pltpu.SemaphoreType.DMA((3,)),
        pltpu.SemaphoreType.DMA((3,)),
    ],
)
def _sc_agg16p(vals_hbm, row_hbm, col_hbm, ew_hbm, zeros_hbm, out_hbm,
               rowbuf, colbuf, ewbuf, qgring, qsring, goring, soring,
               gbuf, sbuf, acc, semg, sems):
  c = lax.axis_index("c")
  s = lax.axis_index("s")
  ebase = (c * NS + s) * E_PER_W
  io = lax.iota(jnp.int32, 16)
  zv = jnp.zeros((16,), jnp.float32)

  pltpu.async_copy(row_hbm.at[pl.ds(ebase, E_PER_W)], rowbuf, semg.at[0])
  pltpu.async_copy(col_hbm.at[pl.ds(ebase, E_PER_W)], colbuf, semg.at[0])
  pltpu.async_copy(ew_hbm.at[pl.ds(ebase, E_PER_W)], ewbuf, semg.at[0])

  for b in range(3):
    sb = sbuf.at[b]

    @pl.loop(0, CHUNK)
    def _(e):
      for j in range(D_IN // 16):
        sb[e, pl.ds(j * 16, 16)] = zv

  @pl.when(s == 0)
  def _():
    pltpu.sync_copy(zeros_hbm, acc)

  for _i in range(3):
    pltpu.make_async_copy(row_hbm.at[pl.ds(ebase, E_PER_W)], rowbuf,
                          semg.at[0]).wait()

  plsc.subcore_barrier()

  def derive(j, q):
    qs2 = qsring.at[q]

    @pl.loop(0, CHUNK, step=16)
    def _(g):
      rv = rowbuf[pl.ds(j * CHUNK + g, 16)]
      qgring[pl.ds(q * CHUNK + g, 16)] = lax.shift_right_logical(rv, 3)
      goring[pl.ds(q * CHUNK + g, 16)] = (rv & 7) * 16
      cv = colbuf[pl.ds(j * CHUNK + g, 16)]
      qs2[pl.ds(g, 16)] = lax.shift_right_logical(cv, 3)
      soring[pl.ds(q * CHUNK + g, 16)] = (cv & 7) * 16

  def g_issue(q):
    pltpu.async_copy(vals_hbm.at[qgring.at[pl.ds(q * CHUNK, CHUNK)]],
                     gbuf.at[q], semg.at[q])

  def g_wait(q):
    pltpu.make_async_copy(vals_hbm.at[qgring.at[pl.ds(q * CHUNK, CHUNK)]],
                          gbuf.at[q], semg.at[q]).wait()

  def s_issue(q):
    pltpu.async_copy(sbuf.at[q], acc.at[qsring.at[q]], sems.at[q], add=True)

  def s_wait_restore(q, restore=True):
    pltpu.make_async_copy(sbuf.at[q], acc.at[qsring.at[q]], sems.at[q]).wait()
    if restore:
      sb = sbuf.at[q]

      @pl.loop(0, CHUNK, step=8)
      def _(g):
        for t in range(8):
          e = g + t
          ev = lax.broadcast(e, (16,))
          sof = plsc.load_gather(soring, [lax.broadcast(q * CHUNK + e, (16,))])
          plsc.store_scatter(sb, [ev, sof + io], zv)

  def scale(i, b):
    gb = gbuf.at[b]
    sb = sbuf.at[b]

    @pl.loop(0, CHUNK, step=8)
    def _(g):
      for t in range(8):
        e = g + t
        ev = lax.broadcast(e, (16,))
        sv = plsc.load_gather(ewbuf, [lax.broadcast(i * CHUNK + e, (16,))])
        gof = plsc.load_gather(goring, [lax.broadcast(b * CHUNK + e, (16,))])
        v = plsc.load_gather(gb, [ev, gof + io]) * sv
        sof = plsc.load_gather(soring, [lax.broadcast(b * CHUNK + e, (16,))])
        plsc.store_scatter(sb, [ev, sof + io], v)

  derive(0, 0)
  g_issue(0)
  derive(1, 1)

  @pl.loop(0, NCH - 2, step=3)
  def _(i0):
    for t in range(3):
      i = i0 + t
      b = t
      bn = (t + 1) % 3
      bp = (t + 2) % 3
      g_wait(b)
      if t == 0:
        @pl.when(i0 >= 1)
        def _():
          s_wait_restore(bp)
      else:
        s_wait_restore(bp)
      derive(i + 2, bp)
      g_issue(bn)
      scale(i, b)
      s_issue(b)

  g_wait(0)
  s_wait_restore(2)
  g_issue(1)
  scale(NCH - 2, 0)
  s_issue(0)

  g_wait(1)
  s_wait_restore(0, restore=False)
  scale(NCH - 1, 1)
  s_issue(1)
  s_wait_restore(1, restore=False)

  plsc.subcore_barrier()

  pltpu.sync_copy(acc.at[pl.ds(s * (NPAD // NS), NPAD // NS)],
                  out_hbm.at[c].at[pl.ds(s * (NPAD // NS), NPAD // NS)])


DEG_ROWS = 80


@functools.partial(
    pl.kernel,
    out_type=jax.ShapeDtypeStruct((NC, DEG_ROWS, D_IN), jnp.float32),
    mesh=_mesh,
    compiler_params=_sc_cp,
    scratch_types=[
        pltpu.VMEM((E_PER_W,), jnp.int32),
        pltpu.VMEM((E_PER_W,), jnp.float32),
        pltpu.VMEM((DEG_ROWS, D_IN), jnp.float32),
        pltpu.VMEM((DEG_ROWS,), jnp.int32),
        pltpu.VMEM_SHARED((DEG_ROWS, D_IN), jnp.float32),
        pltpu.SemaphoreType.DMA,
    ],
)
def _sc_deg(col_hbm, ew_hbm, zeros_hbm, out_hbm, colbuf, ewbuf, degbuf,
            idxbuf, acc, sem0):
  c = lax.axis_index("c")
  s = lax.axis_index("s")
  ebase = (c * NS + s) * E_PER_W
  io = lax.iota(jnp.int32, 16)
  zv = jnp.zeros((16,), jnp.float32)

  pltpu.async_copy(col_hbm.at[pl.ds(ebase, E_PER_W)], colbuf, sem0)
  pltpu.async_copy(ew_hbm.at[pl.ds(ebase, E_PER_W)], ewbuf, sem0)

  @pl.loop(0, DEG_ROWS)
  def _(e):
    for j in range(D_IN // 16):
      degbuf[e, pl.ds(j * 16, 16)] = zv

  @pl.loop(0, DEG_ROWS, step=16)
  def _(g):
    idxbuf[pl.ds(g, 16)] = io + g

  @pl.when(s == 0)
  def _():
    pltpu.sync_copy(zeros_hbm, acc)

  for _i in range(2):
    pltpu.make_async_copy(col_hbm.at[pl.ds(ebase, E_PER_W)], colbuf,
                          sem0).wait()

  plsc.subcore_barrier()

  @pl.loop(0, E_PER_W, step=16)
  def _(g):
    cv = colbuf[pl.ds(g, 16)]
    wv = ewbuf[pl.ds(g, 16)]
    plsc.addupdate_scatter(
        degbuf, [lax.shift_right_logical(cv, 7), cv & 127], wv)

  pltpu.async_copy(degbuf, acc.at[idxbuf], sem0, add=True)
  pltpu.make_async_copy(degbuf, acc.at[idxbuf], sem0).wait()

  plsc.subcore_barrier()

  @pl.when(s == 0)
  def _():
    pltpu.sync_copy(acc, out_hbm.at[c])


_BLK = 1000


def _dis_from(deg0, deg1):
  deg = deg0 + deg1 + 1.0
  return jnp.where(deg > 0, lax.rsqrt(deg), 0.0)


def _tc_prescale_body(deg0_ref, deg1_ref, x_ref, xt_ref):
  dis = _dis_from(deg0_ref[...], deg1_ref[...])
  xt_ref[...] = x_ref[...] * dis


def _tc_mid_body(deg0_ref, deg1_ref, a0_ref, a1_ref, xt_ref, w1_ref, b1_ref,
                 w2_ref, pt_ref):
  dis = _dis_from(deg0_ref[...], deg1_ref[...])
  a = (a0_ref[...] + a1_ref[...] + xt_ref[...]) * dis
  h = jnp.dot(a, w1_ref[...], preferred_element_type=jnp.float32) + b1_ref[...]
  h = jnp.maximum(h, 0.0)
  p = jnp.dot(h, w2_ref[...], preferred_element_type=jnp.float32)
  pt_ref[...] = p * dis


def _tc_final_body(deg0_ref, deg1_ref, a0_ref, a1_ref, pt_ref, b2_ref, out_ref):
  dis = _dis_from(deg0_ref[...], deg1_ref[...])
  t = (a0_ref[...] + a1_ref[...] + pt_ref[...]) * dis
  out_ref[...] = t[:, :D_OUT] + b2_ref[...]


def _nblock(width):
  return pl.BlockSpec((_BLK, width), lambda i: (i, 0))


def _full(shape):
  return pl.BlockSpec(shape, lambda i: tuple(0 for _ in shape))


def kernel(x, edge_index, edge_attr, W1, b1, W2, b2):
  row = edge_index[0]
  col = edge_index[1]
  ew = edge_attr
  z128 = jnp.zeros((N, D_IN), jnp.float32)
  z80 = jnp.zeros((DEG_ROWS, D_IN), jnp.float32)
  z1280 = jnp.zeros((NPAD, D_IN), jnp.float32)
  W2p = jnp.pad(W2, ((0, 0), (0, 16 - D_OUT)))
  b1r = b1.reshape(1, D_HID)
  b2r = b2.reshape(1, D_OUT)

  degP = _sc_deg(col, ew, z80)
  degflat = degP.reshape(NC, DEG_ROWS * D_IN)[:, :N]
  deg0 = degflat[0].reshape(N, 1)
  deg1 = degflat[1].reshape(N, 1)

  xt = pl.pallas_call(
      _tc_prescale_body,
      grid=(N // _BLK,),
      in_specs=[_nblock(1), _nblock(1), _nblock(D_IN)],
      out_specs=_nblock(D_IN),
      out_shape=jax.ShapeDtypeStruct((N, D_IN), jnp.float32),
  )(deg0, deg1, x)

  acc1 = _sc_agg128(xt, row, col, ew, z128)

  pt16 = pl.pallas_call(
      _tc_mid_body,
      grid=(N // _BLK,),
      in_specs=[_nblock(1), _nblock(1), _nblock(D_IN), _nblock(D_IN),
                _nblock(D_IN), _full((D_IN, D_HID)), _full((1, D_HID)),
                _full((D_HID, 16))],
      out_specs=_nblock(16),
      out_shape=jax.ShapeDtypeStruct((N, 16), jnp.float32),
  )(deg0, deg1, acc1[0], acc1[1], xt, W1, b1r, W2p)

  ptPk = pt16.reshape(NP, D_IN)

  acc2P = _sc_agg16p(ptPk, row, col, ew, z1280)
  acc2 = acc2P.reshape(NC, NPAD * 8, 16)[:, :N]

  out = pl.pallas_call(
      _tc_final_body,
      grid=(N // _BLK,),
      in_specs=[_nblock(1), _nblock(1), _nblock(16), _nblock(16),
                _nblock(16), _full((1, D_OUT))],
      out_specs=_nblock(D_OUT),
      out_shape=jax.ShapeDtypeStruct((N, D_OUT), jnp.float32),
  )(deg0, deg1, acc2[0], acc2[1], pt16, b2r)

  return out


# speedup vs baseline: 1.3140x; 1.3060x over previous
import dataclasses
import functools

import jax
import jax.numpy as jnp
from jax import lax
from jax.experimental import pallas as pl
from jax.experimental.pallas import tpu as pltpu
from jax.experimental.pallas import tpu_sc as plsc

N = 10000
E = 320000
D_IN = 128
D_HID = 200
D_OUT = 8

NC = 2
NS = 16
E_PER_W = E // (NC * NS)
CHUNK = 80
NCH = E_PER_W // CHUNK
NBUF = 4
WB_ROWS = 1000
WB_SUBS = N // WB_ROWS

_mesh = plsc.VectorSubcoreMesh(core_axis_name="c", subcore_axis_name="s")

_sc_cp = pltpu.CompilerParams()
if "needs_layout_passes" in pltpu.CompilerParams.__dataclass_fields__:
  _sc_cp = dataclasses.replace(_sc_cp, needs_layout_passes=False)


def _make_sc_agg(nblk):
  NB = 3

  @functools.partial(
      pl.kernel,
      out_type=jax.ShapeDtypeStruct((NC, N, D_IN), jnp.float32),
      mesh=_mesh,
      compiler_params=_sc_cp,
      scratch_types=[
          pltpu.VMEM((NB * CHUNK,), jnp.int32),
          pltpu.VMEM((NB, CHUNK), jnp.int32),
          pltpu.VMEM((NB * CHUNK,), jnp.float32),
          pltpu.VMEM((NB, CHUNK, D_IN), jnp.float32),
          pltpu.VMEM_SHARED((N, D_IN), jnp.float32),
          pltpu.SemaphoreType.DMA((NB,)),
          pltpu.SemaphoreType.DMA((NB,)),
          pltpu.SemaphoreType.DMA((NB,)),
      ],
  )
  def agg(vals_hbm, row_hbm, col_hbm, ew_hbm, zeros_hbm, out_hbm,
          rowbuf, colbuf, ewbuf, gbuf, acc, semi, semg, sems):
    c = lax.axis_index("c")
    s = lax.axis_index("s")
    ebase = (c * NS + s) * E_PER_W

    def idx_issue(j, q):
      off = ebase + j * CHUNK
      pltpu.async_copy(row_hbm.at[pl.ds(off, CHUNK)],
                       rowbuf.at[pl.ds(q * CHUNK, CHUNK)], semi.at[q])
      pltpu.async_copy(col_hbm.at[pl.ds(off, CHUNK)], colbuf.at[q],
                       semi.at[q])
      pltpu.async_copy(ew_hbm.at[pl.ds(off, CHUNK)],
                       ewbuf.at[pl.ds(q * CHUNK, CHUNK)], semi.at[q])

    def idx_wait(q):
      for _ in range(3):
        pltpu.make_async_copy(col_hbm.at[pl.ds(ebase, CHUNK)], colbuf.at[q],
                              semi.at[q]).wait()

    def g_issue(j, q):
      pltpu.async_copy(
          vals_hbm.at[rowbuf.at[pl.ds(q * CHUNK, CHUNK)]], gbuf.at[q],
          semg.at[q])

    def g_wait(q):
      pltpu.make_async_copy(vals_hbm.at[rowbuf.at[pl.ds(q * CHUNK, CHUNK)]],
                            gbuf.at[q], semg.at[q]).wait()

    def s_issue(q):
      pltpu.async_copy(gbuf.at[q], acc.at[colbuf.at[q]], sems.at[q], add=True)

    def s_wait(q):
      pltpu.make_async_copy(gbuf.at[q], acc.at[colbuf.at[q]], sems.at[q]).wait()

    def scale(b):
      gb = gbuf.at[b]

      @pl.loop(0, CHUNK, step=16)
      def _(g):
        for t in range(16):
          e = g + t
          sv = plsc.load_gather(ewbuf, [lax.broadcast(b * CHUNK + e, (16,))])
          for j in range(nblk):
            gb[e, pl.ds(j * 16, 16)] = gb[e, pl.ds(j * 16, 16)] * sv

    @pl.when(s == 0)
    def _():
      pltpu.sync_copy(zeros_hbm, acc)

    plsc.subcore_barrier()

    idx_issue(0, 0)
    idx_issue(1, 1)
    idx_wait(0)
    g_issue(0, 0)

    @pl.loop(0, NCH - 2, step=NB)
    def _(i0):
      for t in range(NB):
        i = i0 + t
        b = t
        bn = (t + 1) % NB
        bp = (t + 2) % NB
        g_wait(b)
        if t == 0:
          @pl.when(i0 >= 1)
          def _():
            s_wait(bp)
        else:
          s_wait(bp)
        idx_wait(bn)
        g_issue(i + 1, bn)
        idx_issue(i + 2, bp)
        scale(b)
        s_issue(b)

    g_wait(0)
    s_wait(2)
    idx_wait(1)
    g_issue(NCH - 1, 1)
    scale(0)
    s_issue(0)

    g_wait(1)
    s_wait(0)
    scale(1)
    s_issue(1)
    s_wait(1)

    plsc.subcore_barrier()

    @pl.when(s < WB_SUBS)
    def _():
      pltpu.sync_copy(acc.at[pl.ds(s * WB_ROWS, WB_ROWS)],
                      out_hbm.at[c].at[pl.ds(s * WB_ROWS, WB_ROWS)])

  return agg


_sc_agg128 = _make_sc_agg(D_IN // 16)
_sc_agg16 = _make_sc_agg(1)

DEG_ROWS = 80


@functools.partial(
    pl.kernel,
    out_type=jax.ShapeDtypeStruct((NC, DEG_ROWS, D_IN), jnp.float32),
    mesh=_mesh,
    compiler_params=_sc_cp,
    scratch_types=[
        pltpu.VMEM((E_PER_W,), jnp.int32),
        pltpu.VMEM((E_PER_W,), jnp.float32),
        pltpu.VMEM((DEG_ROWS, D_IN), jnp.float32),
        pltpu.VMEM((DEG_ROWS,), jnp.int32),
        pltpu.VMEM_SHARED((DEG_ROWS, D_IN), jnp.float32),
        pltpu.SemaphoreType.DMA,
    ],
)
def _sc_deg(col_hbm, ew_hbm, zeros_hbm, out_hbm, colbuf, ewbuf, degbuf,
            idxbuf, acc, sem0):
  c = lax.axis_index("c")
  s = lax.axis_index("s")
  ebase = (c * NS + s) * E_PER_W
  io = lax.iota(jnp.int32, 16)
  zv = jnp.zeros((16,), jnp.float32)

  pltpu.async_copy(col_hbm.at[pl.ds(ebase, E_PER_W)], colbuf, sem0)
  pltpu.async_copy(ew_hbm.at[pl.ds(ebase, E_PER_W)], ewbuf, sem0)

  @pl.loop(0, DEG_ROWS)
  def _(e):
    for j in range(D_IN // 16):
      degbuf[e, pl.ds(j * 16, 16)] = zv

  @pl.loop(0, DEG_ROWS, step=16)
  def _(g):
    idxbuf[pl.ds(g, 16)] = io + g

  @pl.when(s == 0)
  def _():
    pltpu.sync_copy(zeros_hbm, acc)

  for _i in range(2):
    pltpu.make_async_copy(col_hbm.at[pl.ds(ebase, E_PER_W)], colbuf,
                          sem0).wait()

  plsc.subcore_barrier()

  @pl.loop(0, E_PER_W, step=16)
  def _(g):
    cv = colbuf[pl.ds(g, 16)]
    wv = ewbuf[pl.ds(g, 16)]
    plsc.addupdate_scatter(
        degbuf, [lax.shift_right_logical(cv, 7), cv & 127], wv)

  pltpu.async_copy(degbuf, acc.at[idxbuf], sem0, add=True)
  pltpu.make_async_copy(degbuf, acc.at[idxbuf], sem0).wait()

  plsc.subcore_barrier()

  @pl.when(s == 0)
  def _():
    pltpu.sync_copy(acc, out_hbm.at[c])


_BLK = 1000


def _dis_from(deg0, deg1):
  deg = deg0 + deg1 + 1.0
  return jnp.where(deg > 0, lax.rsqrt(deg), 0.0)


def _tc_prescale_body(deg0_ref, deg1_ref, x_ref, xt_ref):
  dis = _dis_from(deg0_ref[...], deg1_ref[...])
  xt_ref[...] = x_ref[...] * dis


def _tc_mid_body(deg0_ref, deg1_ref, a0_ref, a1_ref, xt_ref, w1_ref, b1_ref,
                 w2_ref, pt_ref):
  dis = _dis_from(deg0_ref[...], deg1_ref[...])
  a = (a0_ref[...] + a1_ref[...] + xt_ref[...]) * dis
  h = jnp.dot(a, w1_ref[...], preferred_element_type=jnp.float32) + b1_ref[...]
  h = jnp.maximum(h, 0.0)
  p = jnp.dot(h, w2_ref[...], preferred_element_type=jnp.float32)
  pt_ref[...] = p * dis


def _tc_final_body(deg0_ref, deg1_ref, a0_ref, a1_ref, pt_ref, b2_ref, out_ref):
  dis = _dis_from(deg0_ref[...], deg1_ref[...])
  t = (a0_ref[...] + a1_ref[...] + pt_ref[...]) * dis
  out_ref[...] = t[:, :D_OUT] + b2_ref[...]


def _nblock(width):
  return pl.BlockSpec((_BLK, width), lambda i: (i, 0))


def _full(shape):
  return pl.BlockSpec(shape, lambda i: tuple(0 for _ in shape))


def kernel(x, edge_index, edge_attr, W1, b1, W2, b2):
  row = edge_index[0]
  col = edge_index[1]
  ew = edge_attr
  z128 = jnp.zeros((N, D_IN), jnp.float32)
  z80 = jnp.zeros((DEG_ROWS, D_IN), jnp.float32)
  W2p = jnp.pad(W2, ((0, 0), (0, D_IN - D_OUT)))
  b1r = b1.reshape(1, D_HID)
  b2r = b2.reshape(1, D_OUT)

  degP = _sc_deg(col, ew, z80)
  degflat = degP.reshape(NC, DEG_ROWS * D_IN)[:, :N]
  deg0 = degflat[0].reshape(N, 1)
  deg1 = degflat[1].reshape(N, 1)

  xt = pl.pallas_call(
      _tc_prescale_body,
      grid=(N // _BLK,),
      in_specs=[_nblock(1), _nblock(1), _nblock(D_IN)],
      out_specs=_nblock(D_IN),
      out_shape=jax.ShapeDtypeStruct((N, D_IN), jnp.float32),
  )(deg0, deg1, x)

  acc1 = _sc_agg128(xt, row, col, ew, z128)

  pt = pl.pallas_call(
      _tc_mid_body,
      grid=(N // _BLK,),
      in_specs=[_nblock(1), _nblock(1), _nblock(D_IN), _nblock(D_IN),
                _nblock(D_IN), _full((D_IN, D_HID)), _full((1, D_HID)),
                _full((D_HID, D_IN))],
      out_specs=_nblock(D_IN),
      out_shape=jax.ShapeDtypeStruct((N, D_IN), jnp.float32),
  )(deg0, deg1, acc1[0], acc1[1], xt, W1, b1r, W2p)

  acc2 = _sc_agg16(pt, row, col, ew, z128)

  out = pl.pallas_call(
      _tc_final_body,
      grid=(N // _BLK,),
      in_specs=[_nblock(1), _nblock(1), _nblock(D_IN), _nblock(D_IN),
                _nblock(D_IN), _full((1, D_OUT))],
      out_specs=_nblock(D_OUT),
      out_shape=jax.ShapeDtypeStruct((N, D_OUT), jnp.float32),
  )(deg0, deg1, acc2[0], acc2[1], pt, b2r)

  return out
